# baseline (speedup 1.0000x reference)
import functools

import jax
import jax.numpy as jnp
from jax import lax
from jax.experimental import pallas as pl
from jax.experimental.pallas import tpu as pltpu
from jax.experimental.pallas import tpu_sc as plsc

NUM_EXPERTS = 8
TOP_K = 2
HIDDEN = 1024
FFN = 4096
B, S = 2, 2048
T = B * S

BT = 576
_G_RAW = -(-(T * TOP_K) // BT) + NUM_EXPERTS
G = -(-_G_RAW // 8) * 8
R = G * BT
BF = 1024
NF = FFN // BF

NW = 32
TPW = T // NW
CH_D = 32
CH_C = 16
WR = 128


def _router_body(hs_ref, gwt_ref, pos0_ref, pos1_ref, w0r_ref, w1r_ref,
                 be_ref, act_ref, rlm_ref, rank0_s, rank1_s):
    hs = hs_ref[...]
    logits = lax.dot_general(hs, gwt_ref[...], (((1,), (0,)), ((), ())),
                             preferred_element_type=jnp.float32)
    for b in range(B):
        rlm_ref[b:b + 1, :] = jnp.sum(
            logits[b * S:(b + 1) * S, :], axis=0, keepdims=True) * (1.0 / S)

    m = jnp.max(logits, axis=1, keepdims=True)
    ex = jnp.exp(logits - m)
    probs = ex / jnp.sum(ex, axis=1, keepdims=True)

    eio = lax.broadcasted_iota(jnp.int32, (T, NUM_EXPERTS), 1)
    p0 = jnp.max(probs, axis=1, keepdims=True)
    id0 = jnp.min(jnp.where(probs == p0, eio, NUM_EXPERTS),
                  axis=1, keepdims=True)
    pm = jnp.where(eio == id0, -1.0, probs)
    p1 = jnp.max(pm, axis=1, keepdims=True)
    id1 = jnp.min(jnp.where(pm == p1, eio, NUM_EXPERTS),
                  axis=1, keepdims=True)
    wsum = p0 + p1
    w0 = p0 / wsum
    w1v = p1 / wsum
    oh0 = (eio == id0).astype(jnp.float32)
    oh1 = (eio == id1).astype(jnp.float32)

    rio = lax.broadcasted_iota(jnp.int32, (128, 128), 0)
    cio = lax.broadcasted_iota(jnp.int32, (128, 128), 1)
    lstrict = (cio < rio).astype(jnp.float32)
    carry = jnp.zeros((1, NUM_EXPERTS), jnp.float32)
    for oh, rank_s in ((oh0, rank0_s), (oh1, rank1_s)):
        for bb in range(T // 128):
            blk = oh[bb * 128:(bb + 1) * 128, :]
            rank_s[bb * 128:(bb + 1) * 128, :] = (
                jnp.dot(lstrict, blk, preferred_element_type=jnp.float32)
                + carry)
            carry = carry + jnp.sum(blk, axis=0, keepdims=True)

    counts = carry
    nb = jnp.floor((counts + (BT - 1)) * (1.0 / BT))
    padded = nb * BT
    ei = lax.broadcasted_iota(jnp.int32, (NUM_EXPERTS, NUM_EXPERTS), 0)
    ej = lax.broadcasted_iota(jnp.int32, (NUM_EXPERTS, NUM_EXPERTS), 1)
    mstrict = (ei < ej).astype(jnp.float32)
    mincl = (ei <= ej).astype(jnp.float32)
    offs = jnp.dot(padded, mstrict, preferred_element_type=jnp.float32)
    nbc = jnp.dot(nb, mincl, preferred_element_type=jnp.float32)
    tn = jnp.sum(nb)

    pos0 = jnp.sum(oh0 * (offs + rank0_s[...]), axis=1, keepdims=True)
    pos1 = jnp.sum(oh1 * (offs + rank1_s[...]), axis=1, keepdims=True)
    pos0_ref[...] = pos0.astype(jnp.int32)
    pos1_ref[...] = pos1.astype(jnp.int32)
    ones_w = jnp.ones((1, WR), jnp.float32)
    w0r_ref[...] = w0 * ones_w
    w1r_ref[...] = w1v * ones_w

    gio = lax.broadcasted_iota(jnp.int32, (G, NUM_EXPERTS), 0
                               ).astype(jnp.float32)
    beraw = jnp.sum((nbc <= gio).astype(jnp.float32), axis=1, keepdims=True)
    last_e = jnp.sum((nbc <= (tn - 1.0)).astype(jnp.float32))
    gcol = gio[:, 0:1]
    active = gcol < tn
    be = jnp.where(active, beraw, last_e)
    be_ref[...] = be.astype(jnp.int32)
    act_ref[...] = active.astype(jnp.int32)


def _router(hs, gwt):
    return pl.pallas_call(
        _router_body,
        out_shape=(
            jax.ShapeDtypeStruct((T, 1), jnp.int32),
            jax.ShapeDtypeStruct((T, 1), jnp.int32),
            jax.ShapeDtypeStruct((T, WR), jnp.float32),
            jax.ShapeDtypeStruct((T, WR), jnp.float32),
            jax.ShapeDtypeStruct((G, 1), jnp.int32),
            jax.ShapeDtypeStruct((G, 1), jnp.int32),
            jax.ShapeDtypeStruct((B, NUM_EXPERTS), jnp.float32),
        ),
        scratch_shapes=[
            pltpu.VMEM((T, NUM_EXPERTS), jnp.float32),
            pltpu.VMEM((T, NUM_EXPERTS), jnp.float32),
        ],
    )(hs, gwt)


def _dispatch_body(hs_hbm, pos0_hbm, pos1_hbm, w0r_hbm, w1r_hbm,
                   xs_hbm, ws_hbm, rows_v, wr0_v, wr1_v, idx0_v, idx1_v, sem):
    wid = lax.axis_index("s") * 2 + lax.axis_index("c")
    base = wid * TPW
    for c in range(TPW // CH_D):
        b = base + c * CH_D
        pltpu.sync_copy(pos0_hbm.at[pl.ds(b, CH_D)], idx0_v)
        pltpu.sync_copy(pos1_hbm.at[pl.ds(b, CH_D)], idx1_v)
        pltpu.sync_copy(hs_hbm.at[pl.ds(b, CH_D)], rows_v)
        pltpu.sync_copy(w0r_hbm.at[pl.ds(b, CH_D)], wr0_v)
        pltpu.sync_copy(w1r_hbm.at[pl.ds(b, CH_D)], wr1_v)
        c0 = pltpu.async_copy(rows_v, xs_hbm.at[idx0_v], sem)
        c1 = pltpu.async_copy(rows_v, xs_hbm.at[idx1_v], sem)
        c2 = pltpu.async_copy(wr0_v, ws_hbm.at[idx0_v], sem)
        c3 = pltpu.async_copy(wr1_v, ws_hbm.at[idx1_v], sem)
        c0.wait()
        c1.wait()
        c2.wait()
        c3.wait()


def _dispatch(hs, pos0, pos1, w0r, w1r):
    mesh = plsc.VectorSubcoreMesh(core_axis_name="c", subcore_axis_name="s")
    fn = functools.partial(
        pl.kernel,
        mesh=mesh,
        out_type=(
            jax.ShapeDtypeStruct((R, HIDDEN), jnp.float32),
            jax.ShapeDtypeStruct((R, WR), jnp.float32),
        ),
        scratch_types=[
            pltpu.VMEM((CH_D, HIDDEN), jnp.float32),
            pltpu.VMEM((CH_D, WR), jnp.float32),
            pltpu.VMEM((CH_D, WR), jnp.float32),
            pltpu.VMEM((CH_D,), jnp.int32),
            pltpu.VMEM((CH_D,), jnp.int32),
            pltpu.SemaphoreType.DMA,
        ],
    )(_dispatch_body)
    return fn(hs, pos0, pos1, w0r, w1r)


def _mlp_body(be_ref, act_ref, x_ref, w1_ref, w3_ref, w2_ref, ws_ref, y_ref):
    f = pl.program_id(1)
    g = pl.program_id(0)

    @pl.when(f == 0)
    def _():
        y_ref[...] = jnp.zeros_like(y_ref)

    @pl.when(act_ref[g] > 0)
    def _():
        x = x_ref[...]
        a = lax.dot_general(x, w1_ref[0], (((1,), (1,)), ((), ())),
                            preferred_element_type=jnp.float32)
        bb = lax.dot_general(x, w3_ref[0], (((1,), (1,)), ((), ())),
                             preferred_element_type=jnp.float32)
        h = a * lax.logistic(a) * bb
        y_ref[...] += lax.dot_general(h, w2_ref[0], (((1,), (1,)), ((), ())),
                                      preferred_element_type=jnp.float32)

    @pl.when(f == NF - 1)
    def _():
        y_ref[...] = y_ref[...] * ws_ref[:, 0:1]


def _grouped_mlp(be, act, xs, w1, w3, w2, ws):
    def _fe(g, f):
        return jnp.where((g % 2) == 0, f, NF - 1 - f)

    grid_spec = pltpu.PrefetchScalarGridSpec(
        num_scalar_prefetch=2,
        grid=(G, NF),
        in_specs=[
            pl.BlockSpec((BT, HIDDEN), lambda g, f, be_r, act_r: (g, 0)),
            pl.BlockSpec((1, BF, HIDDEN),
                         lambda g, f, be_r, act_r: (be_r[g], _fe(g, f), 0)),
            pl.BlockSpec((1, BF, HIDDEN),
                         lambda g, f, be_r, act_r: (be_r[g], _fe(g, f), 0)),
            pl.BlockSpec((1, HIDDEN, BF),
                         lambda g, f, be_r, act_r: (be_r[g], 0, _fe(g, f))),
            pl.BlockSpec((BT, WR), lambda g, f, be_r, act_r: (g, 0)),
        ],
        out_specs=pl.BlockSpec((BT, HIDDEN), lambda g, f, be_r, act_r: (g, 0)),
    )
    return pl.pallas_call(
        _mlp_body,
        grid_spec=grid_spec,
        out_shape=jax.ShapeDtypeStruct((R, HIDDEN), jnp.float32),
        compiler_params=pltpu.CompilerParams(
            dimension_semantics=("arbitrary", "arbitrary")),
    )(be, act, xs, w1, w3, w2, ws)


def _combine_body(y_hbm, pos0_hbm, pos1_hbm, out_hbm,
                  r0_v, r1_v, o_v, i0_v, i1_v, sem):
    wid = lax.axis_index("s") * 2 + lax.axis_index("c")
    base = wid * TPW
    for c in range(TPW // CH_C):
        b = base + c * CH_C
        pltpu.sync_copy(pos0_hbm.at[pl.ds(b, CH_C)], i0_v)
        pltpu.sync_copy(pos1_hbm.at[pl.ds(b, CH_C)], i1_v)
        g0 = pltpu.async_copy(y_hbm.at[i0_v], r0_v, sem)
        g1 = pltpu.async_copy(y_hbm.at[i1_v], r1_v, sem)
        g0.wait()
        g1.wait()
        for i in range(CH_C):
            def body(j, _):
                o_v[i, pl.ds(j * 16, 16)] = (
                    r0_v[i, pl.ds(j * 16, 16)] + r1_v[i, pl.ds(j * 16, 16)])
                return 0
            lax.fori_loop(0, HIDDEN // 16, body, 0)
        pltpu.sync_copy(o_v, out_hbm.at[pl.ds(b, CH_C)])


def _combine(y, pos0, pos1):
    mesh = plsc.VectorSubcoreMesh(core_axis_name="c", subcore_axis_name="s")
    fn = functools.partial(
        pl.kernel,
        mesh=mesh,
        out_type=jax.ShapeDtypeStruct((T, HIDDEN), jnp.float32),
        scratch_types=[
            pltpu.VMEM((CH_C, HIDDEN), jnp.float32),
            pltpu.VMEM((CH_C, HIDDEN), jnp.float32),
            pltpu.VMEM((CH_C, HIDDEN), jnp.float32),
            pltpu.VMEM((CH_C,), jnp.int32),
            pltpu.VMEM((CH_C,), jnp.int32),
            pltpu.SemaphoreType.DMA,
        ],
    )(_combine_body)
    return fn(y, pos0, pos1)


def kernel(hidden_states, gate_w, w1, w2, w3):
    hs = hidden_states.reshape(T, HIDDEN)
    pos0, pos1, w0r, w1r, be, act, rlm = _router(hs, gate_w.T)
    pos0f = pos0.reshape(T)
    pos1f = pos1.reshape(T)
    xs, ws = _dispatch(hs, pos0f, pos1f, w0r, w1r)
    y = _grouped_mlp(be.reshape(G), act.reshape(G), xs, w1, w3, w2, ws)
    outf = _combine(y, pos0f, pos1f)
    return outf.reshape(B, S, HIDDEN), rlm


# speedup vs baseline: 1.0531x; 1.0531x over previous
import functools

import jax
import jax.numpy as jnp
from jax import lax
from jax.experimental import pallas as pl
from jax.experimental.pallas import tpu as pltpu
from jax.experimental.pallas import tpu_sc as plsc

NUM_EXPERTS = 8
TOP_K = 2
HIDDEN = 1024
FFN = 4096
B, S = 2, 2048
T = B * S

BT = 544
_G_RAW = -(-(T * TOP_K) // BT) + NUM_EXPERTS
G = -(-_G_RAW // 8) * 8
R = G * BT
BF = 1024
NF = FFN // BF

NW = 32
TPW = T // NW
CH_D = 32
CH_C = 16
WR = 128


def _router_body(hs_ref, gwt_ref, pos0_ref, pos1_ref, w0r_ref, w1r_ref,
                 be_ref, act_ref, rlm_ref, rank0_s, rank1_s):
    hs = hs_ref[...]
    logits = lax.dot_general(hs, gwt_ref[...], (((1,), (0,)), ((), ())),
                             preferred_element_type=jnp.float32)
    for b in range(B):
        rlm_ref[b:b + 1, :] = jnp.sum(
            logits[b * S:(b + 1) * S, :], axis=0, keepdims=True) * (1.0 / S)

    m = jnp.max(logits, axis=1, keepdims=True)
    ex = jnp.exp(logits - m)
    probs = ex / jnp.sum(ex, axis=1, keepdims=True)

    eio = lax.broadcasted_iota(jnp.int32, (T, NUM_EXPERTS), 1)
    p0 = jnp.max(probs, axis=1, keepdims=True)
    id0 = jnp.min(jnp.where(probs == p0, eio, NUM_EXPERTS),
                  axis=1, keepdims=True)
    pm = jnp.where(eio == id0, -1.0, probs)
    p1 = jnp.max(pm, axis=1, keepdims=True)
    id1 = jnp.min(jnp.where(pm == p1, eio, NUM_EXPERTS),
                  axis=1, keepdims=True)
    wsum = p0 + p1
    w0 = p0 / wsum
    w1v = p1 / wsum
    oh0 = (eio == id0).astype(jnp.float32)
    oh1 = (eio == id1).astype(jnp.float32)

    rio = lax.broadcasted_iota(jnp.int32, (128, 128), 0)
    cio = lax.broadcasted_iota(jnp.int32, (128, 128), 1)
    lstrict = (cio < rio).astype(jnp.float32)
    carry = jnp.zeros((1, NUM_EXPERTS), jnp.float32)
    for oh, rank_s in ((oh0, rank0_s), (oh1, rank1_s)):
        for bb in range(T // 128):
            blk = oh[bb * 128:(bb + 1) * 128, :]
            rank_s[bb * 128:(bb + 1) * 128, :] = (
                jnp.dot(lstrict, blk, preferred_element_type=jnp.float32)
                + carry)
            carry = carry + jnp.sum(blk, axis=0, keepdims=True)

    counts = carry
    nb = jnp.floor((counts + (BT - 1)) * (1.0 / BT))
    padded = nb * BT
    ei = lax.broadcasted_iota(jnp.int32, (NUM_EXPERTS, NUM_EXPERTS), 0)
    ej = lax.broadcasted_iota(jnp.int32, (NUM_EXPERTS, NUM_EXPERTS), 1)
    mstrict = (ei < ej).astype(jnp.float32)
    mincl = (ei <= ej).astype(jnp.float32)
    offs = jnp.dot(padded, mstrict, preferred_element_type=jnp.float32)
    nbc = jnp.dot(nb, mincl, preferred_element_type=jnp.float32)
    tn = jnp.sum(nb)

    pos0 = jnp.sum(oh0 * (offs + rank0_s[...]), axis=1, keepdims=True)
    pos1 = jnp.sum(oh1 * (offs + rank1_s[...]), axis=1, keepdims=True)
    pos0_ref[...] = pos0.astype(jnp.int32)
    pos1_ref[...] = pos1.astype(jnp.int32)
    ones_w = jnp.ones((1, WR), jnp.float32)
    w0r_ref[...] = w0 * ones_w
    w1r_ref[...] = w1v * ones_w

    gio = lax.broadcasted_iota(jnp.int32, (G, NUM_EXPERTS), 0
                               ).astype(jnp.float32)
    beraw = jnp.sum((nbc <= gio).astype(jnp.float32), axis=1, keepdims=True)
    last_e = jnp.sum((nbc <= (tn - 1.0)).astype(jnp.float32))
    gcol = gio[:, 0:1]
    active = gcol < tn
    be = jnp.where(active, beraw, last_e)
    be_ref[...] = be.astype(jnp.int32)
    act_ref[...] = active.astype(jnp.int32)


def _router(hs, gwt):
    return pl.pallas_call(
        _router_body,
        out_shape=(
            jax.ShapeDtypeStruct((T, 1), jnp.int32),
            jax.ShapeDtypeStruct((T, 1), jnp.int32),
            jax.ShapeDtypeStruct((T, WR), jnp.float32),
            jax.ShapeDtypeStruct((T, WR), jnp.float32),
            jax.ShapeDtypeStruct((G, 1), jnp.int32),
            jax.ShapeDtypeStruct((G, 1), jnp.int32),
            jax.ShapeDtypeStruct((B, NUM_EXPERTS), jnp.float32),
        ),
        scratch_shapes=[
            pltpu.VMEM((T, NUM_EXPERTS), jnp.float32),
            pltpu.VMEM((T, NUM_EXPERTS), jnp.float32),
        ],
    )(hs, gwt)


def _dispatch_body(hs_hbm, pos0_hbm, pos1_hbm, w0r_hbm, w1r_hbm,
                   xs_hbm, ws_hbm,
                   rows_a, wr0_a, wr1_a, i0_a, i1_a,
                   rows_b, wr0_b, wr1_b, i0_b, i1_b, sem_a, sem_b):
    wid = lax.axis_index("s") * 2 + lax.axis_index("c")
    base = wid * TPW
    bufs = ((rows_a, wr0_a, wr1_a, i0_a, i1_a, sem_a),
            (rows_b, wr0_b, wr1_b, i0_b, i1_b, sem_b))
    nch = TPW // CH_D
    pending = [None, None]
    for c in range(nch):
        rows_v, wr0_v, wr1_v, i0, i1, sem = bufs[c % 2]
        if pending[c % 2] is not None:
            for h in pending[c % 2]:
                h.wait()
        b = base + c * CH_D
        pltpu.sync_copy(pos0_hbm.at[pl.ds(b, CH_D)], i0)
        pltpu.sync_copy(pos1_hbm.at[pl.ds(b, CH_D)], i1)
        pltpu.sync_copy(hs_hbm.at[pl.ds(b, CH_D)], rows_v)
        pltpu.sync_copy(w0r_hbm.at[pl.ds(b, CH_D)], wr0_v)
        pltpu.sync_copy(w1r_hbm.at[pl.ds(b, CH_D)], wr1_v)
        pending[c % 2] = (
            pltpu.async_copy(rows_v, xs_hbm.at[i0], sem),
            pltpu.async_copy(rows_v, xs_hbm.at[i1], sem),
            pltpu.async_copy(wr0_v, ws_hbm.at[i0], sem),
            pltpu.async_copy(wr1_v, ws_hbm.at[i1], sem),
        )
    for p in pending:
        if p is not None:
            for h in p:
                h.wait()


def _dispatch(hs, pos0, pos1, w0r, w1r):
    mesh = plsc.VectorSubcoreMesh(core_axis_name="c", subcore_axis_name="s")
    fn = functools.partial(
        pl.kernel,
        mesh=mesh,
        out_type=(
            jax.ShapeDtypeStruct((R, HIDDEN), jnp.float32),
            jax.ShapeDtypeStruct((R, WR), jnp.float32),
        ),
        scratch_types=[
            pltpu.VMEM((CH_D, HIDDEN), jnp.float32),
            pltpu.VMEM((CH_D, WR), jnp.float32),
            pltpu.VMEM((CH_D, WR), jnp.float32),
            pltpu.VMEM((CH_D,), jnp.int32),
            pltpu.VMEM((CH_D,), jnp.int32),
            pltpu.VMEM((CH_D, HIDDEN), jnp.float32),
            pltpu.VMEM((CH_D, WR), jnp.float32),
            pltpu.VMEM((CH_D, WR), jnp.float32),
            pltpu.VMEM((CH_D,), jnp.int32),
            pltpu.VMEM((CH_D,), jnp.int32),
            pltpu.SemaphoreType.DMA,
            pltpu.SemaphoreType.DMA,
        ],
    )(_dispatch_body)
    return fn(hs, pos0, pos1, w0r, w1r)


def _mlp_body(be_ref, act_ref, x_ref, w1_ref, w3_ref, w2_ref, ws_ref, y_ref):
    f = pl.program_id(1)
    g = pl.program_id(0)

    @pl.when(f == 0)
    def _():
        y_ref[...] = jnp.zeros_like(y_ref)

    @pl.when(act_ref[g] > 0)
    def _():
        x = x_ref[...]
        a = lax.dot_general(x, w1_ref[0], (((1,), (1,)), ((), ())),
                            preferred_element_type=jnp.float32)
        bb = lax.dot_general(x, w3_ref[0], (((1,), (1,)), ((), ())),
                             preferred_element_type=jnp.float32)
        h = a * lax.logistic(a) * bb
        y_ref[...] += lax.dot_general(h, w2_ref[0], (((1,), (1,)), ((), ())),
                                      preferred_element_type=jnp.float32)

    @pl.when(f == NF - 1)
    def _():
        y_ref[...] = y_ref[...] * ws_ref[:, 0:1]


def _grouped_mlp(be, act, xs, w1, w3, w2, ws):
    def _fe(g, f):
        return jnp.where((g % 2) == 0, f, NF - 1 - f)

    grid_spec = pltpu.PrefetchScalarGridSpec(
        num_scalar_prefetch=2,
        grid=(G, NF),
        in_specs=[
            pl.BlockSpec((BT, HIDDEN), lambda g, f, be_r, act_r: (g, 0)),
            pl.BlockSpec((1, BF, HIDDEN),
                         lambda g, f, be_r, act_r: (be_r[g], _fe(g, f), 0)),
            pl.BlockSpec((1, BF, HIDDEN),
                         lambda g, f, be_r, act_r: (be_r[g], _fe(g, f), 0)),
            pl.BlockSpec((1, HIDDEN, BF),
                         lambda g, f, be_r, act_r: (be_r[g], 0, _fe(g, f))),
            pl.BlockSpec((BT, WR), lambda g, f, be_r, act_r: (g, 0)),
        ],
        out_specs=pl.BlockSpec((BT, HIDDEN), lambda g, f, be_r, act_r: (g, 0)),
    )
    return pl.pallas_call(
        _mlp_body,
        grid_spec=grid_spec,
        out_shape=jax.ShapeDtypeStruct((R, HIDDEN), jnp.float32),
        compiler_params=pltpu.CompilerParams(
            dimension_semantics=("arbitrary", "arbitrary")),
    )(be, act, xs, w1, w3, w2, ws)


def _combine_body(y_hbm, pos0_hbm, pos1_hbm, out_hbm,
                  r0_a, r1_a, o_a, i0_a, i1_a,
                  r0_b, r1_b, o_b, i0_b, i1_b,
                  gsem_a, gsem_b, ssem_a, ssem_b):
    wid = lax.axis_index("s") * 2 + lax.axis_index("c")
    base = wid * TPW
    bufs = ((r0_a, r1_a, o_a, i0_a, i1_a, gsem_a, ssem_a),
            (r0_b, r1_b, o_b, i0_b, i1_b, gsem_b, ssem_b))
    nch = TPW // CH_C

    def issue(c):
        r0, r1, _, i0, i1, gs, _ = bufs[c % 2]
        b = base + c * CH_C
        pltpu.sync_copy(pos0_hbm.at[pl.ds(b, CH_C)], i0)
        pltpu.sync_copy(pos1_hbm.at[pl.ds(b, CH_C)], i1)
        return (pltpu.async_copy(y_hbm.at[i0], r0, gs),
                pltpu.async_copy(y_hbm.at[i1], r1, gs))

    pend_g = issue(0)
    pend_s = [None, None]
    for c in range(nch):
        r0, r1, o_v, _, _, _, ss = bufs[c % 2]
        for h in pend_g:
            h.wait()
        if c + 1 < nch:
            pend_g = issue(c + 1)
        if pend_s[c % 2] is not None:
            pend_s[c % 2].wait()
        for i in range(CH_C):
            def body(j, _):
                o_v[i, pl.ds(j * 16, 16)] = (
                    r0[i, pl.ds(j * 16, 16)] + r1[i, pl.ds(j * 16, 16)])
                return 0
            lax.fori_loop(0, HIDDEN // 16, body, 0)
        pend_s[c % 2] = pltpu.async_copy(
            o_v, out_hbm.at[pl.ds(base + c * CH_C, CH_C)], ss)
    for s in pend_s:
        if s is not None:
            s.wait()


def _combine(y, pos0, pos1):
    mesh = plsc.VectorSubcoreMesh(core_axis_name="c", subcore_axis_name="s")
    fn = functools.partial(
        pl.kernel,
        mesh=mesh,
        out_type=jax.ShapeDtypeStruct((T, HIDDEN), jnp.float32),
        scratch_types=[
            pltpu.VMEM((CH_C, HIDDEN), jnp.float32),
            pltpu.VMEM((CH_C, HIDDEN), jnp.float32),
            pltpu.VMEM((CH_C, HIDDEN), jnp.float32),
            pltpu.VMEM((CH_C,), jnp.int32),
            pltpu.VMEM((CH_C,), jnp.int32),
            pltpu.VMEM((CH_C, HIDDEN), jnp.float32),
            pltpu.VMEM((CH_C, HIDDEN), jnp.float32),
            pltpu.VMEM((CH_C, HIDDEN), jnp.float32),
            pltpu.VMEM((CH_C,), jnp.int32),
            pltpu.VMEM((CH_C,), jnp.int32),
            pltpu.SemaphoreType.DMA,
            pltpu.SemaphoreType.DMA,
            pltpu.SemaphoreType.DMA,
            pltpu.SemaphoreType.DMA,
        ],
    )(_combine_body)
    return fn(y, pos0, pos1)


def kernel(hidden_states, gate_w, w1, w2, w3):
    hs = hidden_states.reshape(T, HIDDEN)
    pos0, pos1, w0r, w1r, be, act, rlm = _router(hs, gate_w.T)
    pos0f = pos0.reshape(T)
    pos1f = pos1.reshape(T)
    xs, ws = _dispatch(hs, pos0f, pos1f, w0r, w1r)
    y = _grouped_mlp(be.reshape(G), act.reshape(G), xs, w1, w3, w2, ws)
    outf = _combine(y, pos0f, pos1f)
    return outf.reshape(B, S, HIDDEN), rlm


# speedup vs baseline: 1.1651x; 1.1064x over previous
import functools

import jax
import jax.numpy as jnp
from jax import lax
from jax.experimental import pallas as pl
from jax.experimental.pallas import tpu as pltpu
from jax.experimental.pallas import tpu_sc as plsc

NUM_EXPERTS = 8
TOP_K = 2
HIDDEN = 1024
FFN = 4096
B, S = 2, 2048
T = B * S

BT = 544
_G_RAW = -(-(T * TOP_K) // BT) + NUM_EXPERTS
G = -(-_G_RAW // 8) * 8
R = G * BT
BF = 2048
NF = FFN // BF

NW = 32
TPW = T // NW
CH_D = 32
CH_C = 16
WR = 128


def _router_body(hs_ref, gwt_ref, pos0_ref, pos1_ref, w0r_ref, w1r_ref,
                 be_ref, act_ref, rlm_ref, rank0_s, rank1_s):
    hs = hs_ref[...]
    logits = lax.dot_general(hs, gwt_ref[...], (((1,), (0,)), ((), ())),
                             preferred_element_type=jnp.float32)
    for b in range(B):
        rlm_ref[b:b + 1, :] = jnp.sum(
            logits[b * S:(b + 1) * S, :], axis=0, keepdims=True) * (1.0 / S)

    m = jnp.max(logits, axis=1, keepdims=True)
    ex = jnp.exp(logits - m)
    probs = ex / jnp.sum(ex, axis=1, keepdims=True)

    eio = lax.broadcasted_iota(jnp.int32, (T, NUM_EXPERTS), 1)
    p0 = jnp.max(probs, axis=1, keepdims=True)
    id0 = jnp.min(jnp.where(probs == p0, eio, NUM_EXPERTS),
                  axis=1, keepdims=True)
    pm = jnp.where(eio == id0, -1.0, probs)
    p1 = jnp.max(pm, axis=1, keepdims=True)
    id1 = jnp.min(jnp.where(pm == p1, eio, NUM_EXPERTS),
                  axis=1, keepdims=True)
    wsum = p0 + p1
    w0 = p0 / wsum
    w1v = p1 / wsum
    oh0 = (eio == id0).astype(jnp.float32)
    oh1 = (eio == id1).astype(jnp.float32)

    rio = lax.broadcasted_iota(jnp.int32, (128, 128), 0)
    cio = lax.broadcasted_iota(jnp.int32, (128, 128), 1)
    lstrict = (cio < rio).astype(jnp.float32)
    carry = jnp.zeros((1, NUM_EXPERTS), jnp.float32)
    for oh, rank_s in ((oh0, rank0_s), (oh1, rank1_s)):
        for bb in range(T // 128):
            blk = oh[bb * 128:(bb + 1) * 128, :]
            rank_s[bb * 128:(bb + 1) * 128, :] = (
                jnp.dot(lstrict, blk, preferred_element_type=jnp.float32)
                + carry)
            carry = carry + jnp.sum(blk, axis=0, keepdims=True)

    counts = carry
    nb = jnp.floor((counts + (BT - 1)) * (1.0 / BT))
    padded = nb * BT
    ei = lax.broadcasted_iota(jnp.int32, (NUM_EXPERTS, NUM_EXPERTS), 0)
    ej = lax.broadcasted_iota(jnp.int32, (NUM_EXPERTS, NUM_EXPERTS), 1)
    mstrict = (ei < ej).astype(jnp.float32)
    mincl = (ei <= ej).astype(jnp.float32)
    offs = jnp.dot(padded, mstrict, preferred_element_type=jnp.float32)
    nbc = jnp.dot(nb, mincl, preferred_element_type=jnp.float32)
    tn = jnp.sum(nb)

    pos0 = jnp.sum(oh0 * (offs + rank0_s[...]), axis=1, keepdims=True)
    pos1 = jnp.sum(oh1 * (offs + rank1_s[...]), axis=1, keepdims=True)
    pos0_ref[...] = pos0.astype(jnp.int32)
    pos1_ref[...] = pos1.astype(jnp.int32)
    ones_w = jnp.ones((1, WR), jnp.float32)
    w0r_ref[...] = w0 * ones_w
    w1r_ref[...] = w1v * ones_w

    gio = lax.broadcasted_iota(jnp.int32, (G, NUM_EXPERTS), 0
                               ).astype(jnp.float32)
    beraw = jnp.sum((nbc <= gio).astype(jnp.float32), axis=1, keepdims=True)
    last_e = jnp.sum((nbc <= (tn - 1.0)).astype(jnp.float32))
    gcol = gio[:, 0:1]
    active = gcol < tn
    be = jnp.where(active, beraw, last_e)
    be_ref[...] = be.astype(jnp.int32)
    act_ref[...] = active.astype(jnp.int32)


def _router(hs, gwt):
    return pl.pallas_call(
        _router_body,
        out_shape=(
            jax.ShapeDtypeStruct((T, 1), jnp.int32),
            jax.ShapeDtypeStruct((T, 1), jnp.int32),
            jax.ShapeDtypeStruct((T, WR), jnp.float32),
            jax.ShapeDtypeStruct((T, WR), jnp.float32),
            jax.ShapeDtypeStruct((G, 1), jnp.int32),
            jax.ShapeDtypeStruct((G, 1), jnp.int32),
            jax.ShapeDtypeStruct((B, NUM_EXPERTS), jnp.float32),
        ),
        scratch_shapes=[
            pltpu.VMEM((T, NUM_EXPERTS), jnp.float32),
            pltpu.VMEM((T, NUM_EXPERTS), jnp.float32),
        ],
    )(hs, gwt)


def _dispatch_body(hs_hbm, pos0_hbm, pos1_hbm, w0r_hbm, w1r_hbm,
                   xs_hbm, ws_hbm,
                   rows_a, wr0_a, wr1_a, i0_a, i1_a,
                   rows_b, wr0_b, wr1_b, i0_b, i1_b, sem_a, sem_b):
    wid = lax.axis_index("s") * 2 + lax.axis_index("c")
    base = wid * TPW
    bufs = ((rows_a, wr0_a, wr1_a, i0_a, i1_a, sem_a),
            (rows_b, wr0_b, wr1_b, i0_b, i1_b, sem_b))
    nch = TPW // CH_D
    pending = [None, None]
    for c in range(nch):
        rows_v, wr0_v, wr1_v, i0, i1, sem = bufs[c % 2]
        if pending[c % 2] is not None:
            for h in pending[c % 2]:
                h.wait()
        b = base + c * CH_D
        pltpu.sync_copy(pos0_hbm.at[pl.ds(b, CH_D)], i0)
        pltpu.sync_copy(pos1_hbm.at[pl.ds(b, CH_D)], i1)
        pltpu.sync_copy(hs_hbm.at[pl.ds(b, CH_D)], rows_v)
        pltpu.sync_copy(w0r_hbm.at[pl.ds(b, CH_D)], wr0_v)
        pltpu.sync_copy(w1r_hbm.at[pl.ds(b, CH_D)], wr1_v)
        pending[c % 2] = (
            pltpu.async_copy(rows_v, xs_hbm.at[i0], sem),
            pltpu.async_copy(rows_v, xs_hbm.at[i1], sem),
            pltpu.async_copy(wr0_v, ws_hbm.at[i0], sem),
            pltpu.async_copy(wr1_v, ws_hbm.at[i1], sem),
        )
    for p in pending:
        if p is not None:
            for h in p:
                h.wait()


def _dispatch(hs, pos0, pos1, w0r, w1r):
    mesh = plsc.VectorSubcoreMesh(core_axis_name="c", subcore_axis_name="s")
    fn = functools.partial(
        pl.kernel,
        mesh=mesh,
        out_type=(
            jax.ShapeDtypeStruct((R, HIDDEN), jnp.float32),
            jax.ShapeDtypeStruct((R, WR), jnp.float32),
        ),
        scratch_types=[
            pltpu.VMEM((CH_D, HIDDEN), jnp.float32),
            pltpu.VMEM((CH_D, WR), jnp.float32),
            pltpu.VMEM((CH_D, WR), jnp.float32),
            pltpu.VMEM((CH_D,), jnp.int32),
            pltpu.VMEM((CH_D,), jnp.int32),
            pltpu.VMEM((CH_D, HIDDEN), jnp.float32),
            pltpu.VMEM((CH_D, WR), jnp.float32),
            pltpu.VMEM((CH_D, WR), jnp.float32),
            pltpu.VMEM((CH_D,), jnp.int32),
            pltpu.VMEM((CH_D,), jnp.int32),
            pltpu.SemaphoreType.DMA,
            pltpu.SemaphoreType.DMA,
        ],
    )(_dispatch_body)
    return fn(hs, pos0, pos1, w0r, w1r)


def _mlp_body(be_ref, act_ref, x_ref, w1_ref, w3_ref, w2_ref, ws_ref, y_ref):
    f = pl.program_id(1)
    g = pl.program_id(0)

    @pl.when(f == 0)
    def _():
        y_ref[...] = jnp.zeros_like(y_ref)

    @pl.when(act_ref[g] > 0)
    def _():
        x = x_ref[...]
        a = lax.dot_general(x, w1_ref[0], (((1,), (1,)), ((), ())),
                            preferred_element_type=jnp.float32)
        bb = lax.dot_general(x, w3_ref[0], (((1,), (1,)), ((), ())),
                             preferred_element_type=jnp.float32)
        h = a * lax.logistic(a) * bb
        y_ref[...] += lax.dot_general(h, w2_ref[0], (((1,), (1,)), ((), ())),
                                      preferred_element_type=jnp.float32)

    @pl.when(f == NF - 1)
    def _():
        y_ref[...] = y_ref[...] * ws_ref[:, 0:1]


def _grouped_mlp(be, act, xs, w1, w3, w2, ws):
    def _fe(g, f):
        return jnp.where((g % 2) == 0, f, NF - 1 - f)

    grid_spec = pltpu.PrefetchScalarGridSpec(
        num_scalar_prefetch=2,
        grid=(G, NF),
        in_specs=[
            pl.BlockSpec((BT, HIDDEN), lambda g, f, be_r, act_r: (g, 0)),
            pl.BlockSpec((1, BF, HIDDEN),
                         lambda g, f, be_r, act_r: (be_r[g], _fe(g, f), 0)),
            pl.BlockSpec((1, BF, HIDDEN),
                         lambda g, f, be_r, act_r: (be_r[g], _fe(g, f), 0)),
            pl.BlockSpec((1, HIDDEN, BF),
                         lambda g, f, be_r, act_r: (be_r[g], 0, _fe(g, f))),
            pl.BlockSpec((BT, WR), lambda g, f, be_r, act_r: (g, 0)),
        ],
        out_specs=pl.BlockSpec((BT, HIDDEN), lambda g, f, be_r, act_r: (g, 0)),
    )
    return pl.pallas_call(
        _mlp_body,
        grid_spec=grid_spec,
        out_shape=jax.ShapeDtypeStruct((R, HIDDEN), jnp.float32),
        compiler_params=pltpu.CompilerParams(
            dimension_semantics=("arbitrary", "arbitrary"),
            vmem_limit_bytes=100 * 1024 * 1024),
    )(be, act, xs, w1, w3, w2, ws)


def _combine_body(y_hbm, pos0_hbm, pos1_hbm, out_hbm,
                  r0_a, r1_a, o_a, i0_a, i1_a,
                  r0_b, r1_b, o_b, i0_b, i1_b,
                  gsem_a, gsem_b, ssem_a, ssem_b):
    wid = lax.axis_index("s") * 2 + lax.axis_index("c")
    base = wid * TPW
    bufs = ((r0_a, r1_a, o_a, i0_a, i1_a, gsem_a, ssem_a),
            (r0_b, r1_b, o_b, i0_b, i1_b, gsem_b, ssem_b))
    nch = TPW // CH_C

    def issue(c):
        r0, r1, _, i0, i1, gs, _ = bufs[c % 2]
        b = base + c * CH_C
        pltpu.sync_copy(pos0_hbm.at[pl.ds(b, CH_C)], i0)
        pltpu.sync_copy(pos1_hbm.at[pl.ds(b, CH_C)], i1)
        return (pltpu.async_copy(y_hbm.at[i0], r0, gs),
                pltpu.async_copy(y_hbm.at[i1], r1, gs))

    pend_g = issue(0)
    pend_s = [None, None]
    for c in range(nch):
        r0, r1, o_v, _, _, _, ss = bufs[c % 2]
        for h in pend_g:
            h.wait()
        if c + 1 < nch:
            pend_g = issue(c + 1)
        if pend_s[c % 2] is not None:
            pend_s[c % 2].wait()
        for i in range(CH_C):
            def body(j, _):
                o_v[i, pl.ds(j * 16, 16)] = (
                    r0[i, pl.ds(j * 16, 16)] + r1[i, pl.ds(j * 16, 16)])
                return 0
            lax.fori_loop(0, HIDDEN // 16, body, 0)
        pend_s[c % 2] = pltpu.async_copy(
            o_v, out_hbm.at[pl.ds(base + c * CH_C, CH_C)], ss)
    for s in pend_s:
        if s is not None:
            s.wait()


def _combine(y, pos0, pos1):
    mesh = plsc.VectorSubcoreMesh(core_axis_name="c", subcore_axis_name="s")
    fn = functools.partial(
        pl.kernel,
        mesh=mesh,
        out_type=jax.ShapeDtypeStruct((T, HIDDEN), jnp.float32),
        scratch_types=[
            pltpu.VMEM((CH_C, HIDDEN), jnp.float32),
            pltpu.VMEM((CH_C, HIDDEN), jnp.float32),
            pltpu.VMEM((CH_C, HIDDEN), jnp.float32),
            pltpu.VMEM((CH_C,), jnp.int32),
            pltpu.VMEM((CH_C,), jnp.int32),
            pltpu.VMEM((CH_C, HIDDEN), jnp.float32),
            pltpu.VMEM((CH_C, HIDDEN), jnp.float32),
            pltpu.VMEM((CH_C, HIDDEN), jnp.float32),
            pltpu.VMEM((CH_C,), jnp.int32),
            pltpu.VMEM((CH_C,), jnp.int32),
            pltpu.SemaphoreType.DMA,
            pltpu.SemaphoreType.DMA,
            pltpu.SemaphoreType.DMA,
            pltpu.SemaphoreType.DMA,
        ],
    )(_combine_body)
    return fn(y, pos0, pos1)


def kernel(hidden_states, gate_w, w1, w2, w3):
    hs = hidden_states.reshape(T, HIDDEN)
    pos0, pos1, w0r, w1r, be, act, rlm = _router(hs, gate_w.T)
    pos0f = pos0.reshape(T)
    pos1f = pos1.reshape(T)
    xs, ws = _dispatch(hs, pos0f, pos1f, w0r, w1r)
    y = _grouped_mlp(be.reshape(G), act.reshape(G), xs, w1, w3, w2, ws)
    outf = _combine(y, pos0f, pos1f)
    return outf.reshape(B, S, HIDDEN), rlm


# speedup vs baseline: 1.1720x; 1.0058x over previous
import functools

import jax
import jax.numpy as jnp
from jax import lax
from jax.experimental import pallas as pl
from jax.experimental.pallas import tpu as pltpu
from jax.experimental.pallas import tpu_sc as plsc

NUM_EXPERTS = 8
TOP_K = 2
HIDDEN = 1024
FFN = 4096
B, S = 2, 2048
T = B * S

BT = 544
_G_RAW = -(-(T * TOP_K) // BT) + NUM_EXPERTS
G = -(-_G_RAW // 8) * 8
R = G * BT
BF = 2048
NF = FFN // BF

NW = 32
TPW = T // NW
CH_D = 32
CH_C = 16
WR = 128


def _router_body(hs_ref, gwt_ref, pos0_ref, pos1_ref, w0r_ref, w1r_ref,
                 be_ref, act_ref, rlm_ref, rank0_s, rank1_s):
    hs = hs_ref[...]
    logits = lax.dot_general(hs, gwt_ref[...], (((1,), (0,)), ((), ())),
                             preferred_element_type=jnp.float32)
    for b in range(B):
        rlm_ref[b:b + 1, :] = jnp.sum(
            logits[b * S:(b + 1) * S, :], axis=0, keepdims=True) * (1.0 / S)

    m = jnp.max(logits, axis=1, keepdims=True)
    ex = jnp.exp(logits - m)
    probs = ex / jnp.sum(ex, axis=1, keepdims=True)

    eio = lax.broadcasted_iota(jnp.int32, (T, NUM_EXPERTS), 1)
    p0 = jnp.max(probs, axis=1, keepdims=True)
    id0 = jnp.min(jnp.where(probs == p0, eio, NUM_EXPERTS),
                  axis=1, keepdims=True)
    pm = jnp.where(eio == id0, -1.0, probs)
    p1 = jnp.max(pm, axis=1, keepdims=True)
    id1 = jnp.min(jnp.where(pm == p1, eio, NUM_EXPERTS),
                  axis=1, keepdims=True)
    wsum = p0 + p1
    w0 = p0 / wsum
    w1v = p1 / wsum
    oh0 = (eio == id0).astype(jnp.float32)
    oh1 = (eio == id1).astype(jnp.float32)

    rio = lax.broadcasted_iota(jnp.int32, (128, 128), 0)
    cio = lax.broadcasted_iota(jnp.int32, (128, 128), 1)
    lstrict = (cio < rio).astype(jnp.float32)
    carry = jnp.zeros((1, NUM_EXPERTS), jnp.float32)
    for oh, rank_s in ((oh0, rank0_s), (oh1, rank1_s)):
        for bb in range(T // 128):
            blk = oh[bb * 128:(bb + 1) * 128, :]
            rank_s[bb * 128:(bb + 1) * 128, :] = (
                jnp.dot(lstrict, blk, preferred_element_type=jnp.float32)
                + carry)
            carry = carry + jnp.sum(blk, axis=0, keepdims=True)

    counts = carry
    nb = jnp.floor((counts + (BT - 1)) * (1.0 / BT))
    padded = nb * BT
    ei = lax.broadcasted_iota(jnp.int32, (NUM_EXPERTS, NUM_EXPERTS), 0)
    ej = lax.broadcasted_iota(jnp.int32, (NUM_EXPERTS, NUM_EXPERTS), 1)
    mstrict = (ei < ej).astype(jnp.float32)
    mincl = (ei <= ej).astype(jnp.float32)
    offs = jnp.dot(padded, mstrict, preferred_element_type=jnp.float32)
    nbc = jnp.dot(nb, mincl, preferred_element_type=jnp.float32)
    tn = jnp.sum(nb)

    pos0 = jnp.sum(oh0 * (offs + rank0_s[...]), axis=1, keepdims=True)
    pos1 = jnp.sum(oh1 * (offs + rank1_s[...]), axis=1, keepdims=True)
    pos0_ref[...] = pos0.astype(jnp.int32)
    pos1_ref[...] = pos1.astype(jnp.int32)
    ones_w = jnp.ones((1, WR), jnp.float32)
    w0r_ref[...] = w0 * ones_w
    w1r_ref[...] = w1v * ones_w

    gio = lax.broadcasted_iota(jnp.int32, (G, NUM_EXPERTS), 0
                               ).astype(jnp.float32)
    beraw = jnp.sum((nbc <= gio).astype(jnp.float32), axis=1, keepdims=True)
    last_e = jnp.sum((nbc <= (tn - 1.0)).astype(jnp.float32))
    gcol = gio[:, 0:1]
    active = gcol < tn
    be = jnp.where(active, beraw, last_e)
    be_ref[...] = be.astype(jnp.int32)
    act_ref[...] = active.astype(jnp.int32)


def _router(hs, gwt):
    return pl.pallas_call(
        _router_body,
        out_shape=(
            jax.ShapeDtypeStruct((T, 1), jnp.int32),
            jax.ShapeDtypeStruct((T, 1), jnp.int32),
            jax.ShapeDtypeStruct((T, WR), jnp.float32),
            jax.ShapeDtypeStruct((T, WR), jnp.float32),
            jax.ShapeDtypeStruct((G, 1), jnp.int32),
            jax.ShapeDtypeStruct((G, 1), jnp.int32),
            jax.ShapeDtypeStruct((B, NUM_EXPERTS), jnp.float32),
        ),
        scratch_shapes=[
            pltpu.VMEM((T, NUM_EXPERTS), jnp.float32),
            pltpu.VMEM((T, NUM_EXPERTS), jnp.float32),
        ],
    )(hs, gwt)


def _dispatch_body(hs_hbm, pos0_hbm, pos1_hbm, w0r_hbm, w1r_hbm,
                   xs_hbm, ws_hbm,
                   rows_a, wr0_a, wr1_a, i0_a, i1_a,
                   rows_b, wr0_b, wr1_b, i0_b, i1_b, sem_a, sem_b):
    wid = lax.axis_index("s") * 2 + lax.axis_index("c")
    base = wid * TPW
    bufs = ((rows_a, wr0_a, wr1_a, i0_a, i1_a, sem_a),
            (rows_b, wr0_b, wr1_b, i0_b, i1_b, sem_b))
    nch = TPW // CH_D
    pending = [None, None]
    for c in range(nch):
        rows_v, wr0_v, wr1_v, i0, i1, sem = bufs[c % 2]
        if pending[c % 2] is not None:
            for h in pending[c % 2]:
                h.wait()
        b = base + c * CH_D
        pltpu.sync_copy(pos0_hbm.at[pl.ds(b, CH_D)], i0)
        pltpu.sync_copy(pos1_hbm.at[pl.ds(b, CH_D)], i1)
        pltpu.sync_copy(hs_hbm.at[pl.ds(b, CH_D)], rows_v)
        pltpu.sync_copy(w0r_hbm.at[pl.ds(b, CH_D)], wr0_v)
        pltpu.sync_copy(w1r_hbm.at[pl.ds(b, CH_D)], wr1_v)
        pending[c % 2] = (
            pltpu.async_copy(rows_v, xs_hbm.at[i0], sem),
            pltpu.async_copy(rows_v, xs_hbm.at[i1], sem),
            pltpu.async_copy(wr0_v, ws_hbm.at[i0], sem),
            pltpu.async_copy(wr1_v, ws_hbm.at[i1], sem),
        )
    for p in pending:
        if p is not None:
            for h in p:
                h.wait()


def _dispatch(hs, pos0, pos1, w0r, w1r):
    mesh = plsc.VectorSubcoreMesh(core_axis_name="c", subcore_axis_name="s")
    fn = functools.partial(
        pl.kernel,
        mesh=mesh,
        out_type=(
            jax.ShapeDtypeStruct((R, HIDDEN), jnp.float32),
            jax.ShapeDtypeStruct((R, WR), jnp.float32),
        ),
        scratch_types=[
            pltpu.VMEM((CH_D, HIDDEN), jnp.float32),
            pltpu.VMEM((CH_D, WR), jnp.float32),
            pltpu.VMEM((CH_D, WR), jnp.float32),
            pltpu.VMEM((CH_D,), jnp.int32),
            pltpu.VMEM((CH_D,), jnp.int32),
            pltpu.VMEM((CH_D, HIDDEN), jnp.float32),
            pltpu.VMEM((CH_D, WR), jnp.float32),
            pltpu.VMEM((CH_D, WR), jnp.float32),
            pltpu.VMEM((CH_D,), jnp.int32),
            pltpu.VMEM((CH_D,), jnp.int32),
            pltpu.SemaphoreType.DMA,
            pltpu.SemaphoreType.DMA,
        ],
    )(_dispatch_body)
    return fn(hs, pos0, pos1, w0r, w1r)


def _mlp_body(be_ref, act_ref, x_ref, w1_ref, w3_ref, w2_ref, ws_ref, y_ref):
    f = pl.program_id(1)
    g = pl.program_id(0)

    @pl.when(f == 0)
    def _():
        y_ref[...] = jnp.zeros_like(y_ref)

    @pl.when(act_ref[g] > 0)
    def _():
        x = x_ref[...]
        a = lax.dot_general(x, w1_ref[0], (((1,), (1,)), ((), ())),
                            preferred_element_type=jnp.float32)
        bb = lax.dot_general(x, w3_ref[0], (((1,), (1,)), ((), ())),
                             preferred_element_type=jnp.float32)
        h = a * lax.logistic(a) * bb
        y_ref[...] += lax.dot_general(h, w2_ref[0], (((1,), (1,)), ((), ())),
                                      preferred_element_type=jnp.float32)

    @pl.when(f == NF - 1)
    def _():
        y_ref[...] = y_ref[...] * ws_ref[:, 0:1]


def _grouped_mlp(be, act, xs, w1, w3, w2, ws):
    def _fe(g, f):
        return jnp.where((g % 2) == 0, f, NF - 1 - f)

    grid_spec = pltpu.PrefetchScalarGridSpec(
        num_scalar_prefetch=2,
        grid=(G, NF),
        in_specs=[
            pl.BlockSpec((BT, HIDDEN), lambda g, f, be_r, act_r: (g, 0)),
            pl.BlockSpec((1, BF, HIDDEN),
                         lambda g, f, be_r, act_r: (be_r[g], _fe(g, f), 0)),
            pl.BlockSpec((1, BF, HIDDEN),
                         lambda g, f, be_r, act_r: (be_r[g], _fe(g, f), 0)),
            pl.BlockSpec((1, HIDDEN, BF),
                         lambda g, f, be_r, act_r: (be_r[g], 0, _fe(g, f))),
            pl.BlockSpec((BT, WR), lambda g, f, be_r, act_r: (g, 0)),
        ],
        out_specs=pl.BlockSpec((BT, HIDDEN), lambda g, f, be_r, act_r: (g, 0)),
    )
    return pl.pallas_call(
        _mlp_body,
        grid_spec=grid_spec,
        out_shape=jax.ShapeDtypeStruct((R, HIDDEN), jnp.float32),
        compiler_params=pltpu.CompilerParams(
            dimension_semantics=("arbitrary", "arbitrary"),
            vmem_limit_bytes=127 * 1024 * 1024),
    )(be, act, xs, w1, w3, w2, ws)


def _combine_body(y_hbm, pos0_hbm, pos1_hbm, out_hbm,
                  r0_a, r1_a, o_a, i0_a, i1_a,
                  r0_b, r1_b, o_b, i0_b, i1_b,
                  gsem_a, gsem_b, ssem_a, ssem_b):
    wid = lax.axis_index("s") * 2 + lax.axis_index("c")
    base = wid * TPW
    bufs = ((r0_a, r1_a, o_a, i0_a, i1_a, gsem_a, ssem_a),
            (r0_b, r1_b, o_b, i0_b, i1_b, gsem_b, ssem_b))
    nch = TPW // CH_C

    def issue(c):
        r0, r1, _, i0, i1, gs, _ = bufs[c % 2]
        b = base + c * CH_C
        pltpu.sync_copy(pos0_hbm.at[pl.ds(b, CH_C)], i0)
        pltpu.sync_copy(pos1_hbm.at[pl.ds(b, CH_C)], i1)
        return (pltpu.async_copy(y_hbm.at[i0], r0, gs),
                pltpu.async_copy(y_hbm.at[i1], r1, gs))

    pend_g = issue(0)
    pend_s = [None, None]
    for c in range(nch):
        r0, r1, o_v, _, _, _, ss = bufs[c % 2]
        for h in pend_g:
            h.wait()
        if c + 1 < nch:
            pend_g = issue(c + 1)
        if pend_s[c % 2] is not None:
            pend_s[c % 2].wait()
        for i in range(CH_C):
            def body(j, _):
                o_v[i, pl.ds(j * 16, 16)] = (
                    r0[i, pl.ds(j * 16, 16)] + r1[i, pl.ds(j * 16, 16)])
                return 0
            lax.fori_loop(0, HIDDEN // 16, body, 0, unroll=2)
        pend_s[c % 2] = pltpu.async_copy(
            o_v, out_hbm.at[pl.ds(base + c * CH_C, CH_C)], ss)
    for s in pend_s:
        if s is not None:
            s.wait()


def _combine(y, pos0, pos1):
    mesh = plsc.VectorSubcoreMesh(core_axis_name="c", subcore_axis_name="s")
    fn = functools.partial(
        pl.kernel,
        mesh=mesh,
        out_type=jax.ShapeDtypeStruct((T, HIDDEN), jnp.float32),
        scratch_types=[
            pltpu.VMEM((CH_C, HIDDEN), jnp.float32),
            pltpu.VMEM((CH_C, HIDDEN), jnp.float32),
            pltpu.VMEM((CH_C, HIDDEN), jnp.float32),
            pltpu.VMEM((CH_C,), jnp.int32),
            pltpu.VMEM((CH_C,), jnp.int32),
            pltpu.VMEM((CH_C, HIDDEN), jnp.float32),
            pltpu.VMEM((CH_C, HIDDEN), jnp.float32),
            pltpu.VMEM((CH_C, HIDDEN), jnp.float32),
            pltpu.VMEM((CH_C,), jnp.int32),
            pltpu.VMEM((CH_C,), jnp.int32),
            pltpu.SemaphoreType.DMA,
            pltpu.SemaphoreType.DMA,
            pltpu.SemaphoreType.DMA,
            pltpu.SemaphoreType.DMA,
        ],
    )(_combine_body)
    return fn(y, pos0, pos1)


def kernel(hidden_states, gate_w, w1, w2, w3):
    hs = hidden_states.reshape(T, HIDDEN)
    pos0, pos1, w0r, w1r, be, act, rlm = _router(hs, gate_w.T)
    pos0f = pos0.reshape(T)
    pos1f = pos1.reshape(T)
    xs, ws = _dispatch(hs, pos0f, pos1f, w0r, w1r)
    y = _grouped_mlp(be.reshape(G), act.reshape(G), xs, w1, w3, w2, ws)
    outf = _combine(y, pos0f, pos1f)
    return outf.reshape(B, S, HIDDEN), rlm
